# baseline (speedup 1.0000x reference)
import functools

import jax
import jax.numpy as jnp
import numpy as np
from jax import lax
from jax.experimental import pallas as pl
from jax.experimental.pallas import tpu as pltpu
from jax.experimental.pallas import tpu_sc as plsc

_VARIABLES = ["tp", "cp", "swl"]
_DATA_INDEX = np.array([3, 17, 42], dtype=np.int64)
_MIN_VAL = [0.0, 0.0, 0.0]
_NORMALIZER = ["mean-std", "min-max", "std"]
_STATS = {
    "mean": np.array([0.5, 0.2, 0.1], dtype=np.float32),
    "stdev": np.array([1.2, 0.8, 0.3], dtype=np.float32),
    "min": np.array([0.0, 0.0, 0.0], dtype=np.float32),
    "max": np.array([10.0, 5.0, 1.0], dtype=np.float32),
}
_NAME_TO_INDEX_STATS = {"tp": 0, "cp": 1, "swl": 2}


def _compute_norm_min_val() -> np.ndarray:
    nmv = np.zeros(len(_VARIABLES), dtype=np.float32)
    for ii, var in enumerate(_VARIABLES):
        si = _NAME_TO_INDEX_STATS[var]
        if _NORMALIZER[ii] == "mean-std":
            nmv[ii] = (_MIN_VAL[ii] - _STATS["mean"][si]) / _STATS["stdev"][si]
        elif _NORMALIZER[ii] == "min-max":
            nmv[ii] = (_MIN_VAL[ii] - _STATS["min"][si]) / (_STATS["max"][si] - _STATS["min"][si])
        elif _NORMALIZER[ii] == "max":
            nmv[ii] = _MIN_VAL[ii] / _STATS["max"][si]
        elif _NORMALIZER[ii] == "std":
            nmv[ii] = _MIN_VAL[ii] / _STATS["stdev"][si]
    return nmv


_NMV = _compute_norm_min_val()


def _bound_kernel(x_ref, o_ref):
    lane = jax.lax.broadcasted_iota(jnp.int32, (1, 128), 1)
    lmin = jnp.full((1, 128), -jnp.inf, jnp.float32)
    for c, v in zip(_DATA_INDEX, _NMV):
        lmin = jnp.where(lane == int(c), jnp.float32(v), lmin)
    o_ref[...] = jnp.maximum(x_ref[...], lmin)


@functools.partial(jax.jit, static_argnames=("block_rows",))
def _bound(x2d, block_rows):
    rows = x2d.shape[0]
    return pl.pallas_call(
        _bound_kernel,
        grid=(rows // block_rows,),
        in_specs=[pl.BlockSpec((block_rows, 128), lambda i: (i, 0))],
        out_specs=pl.BlockSpec((block_rows, 128), lambda i: (i, 0)),
        out_shape=jax.ShapeDtypeStruct(x2d.shape, x2d.dtype),
    )(x2d)


@jax.jit
def _sc_bound(x1d):
    info = plsc.get_sparse_core_info()
    nc, ns = info.num_cores, info.num_subcores
    nw = nc * ns
    total = x1d.shape[0]
    per_w = total // nw
    chunk = 32768
    n_chunks = per_w // chunk
    mesh = plsc.VectorSubcoreMesh(core_axis_name="c", subcore_axis_name="s")

    @functools.partial(
        pl.kernel,
        mesh=mesh,
        out_type=jax.ShapeDtypeStruct((total,), jnp.float32),
        scratch_types=[pltpu.VMEM((chunk,), jnp.float32)],
    )
    def k(x_hbm, o_hbm, buf):
        wid = lax.axis_index("s") * nc + lax.axis_index("c")
        base = wid * per_w
        lane = lax.iota(jnp.int32, 16)
        ninf = jnp.float32(-jnp.inf)
        floors = [
            (0, jnp.where(lane == 3, jnp.float32(_NMV[0]), ninf)),
            (16, jnp.where(lane == 1, jnp.float32(_NMV[1]), ninf)),
            (32, jnp.where(lane == 10, jnp.float32(_NMV[2]), ninf)),
        ]

        def fix_row(r, carry):
            row = r * 128
            for off, fl in floors:
                sl = pl.ds(row + off, 16)
                buf[sl] = jnp.maximum(buf[sl], fl)
            return carry

        def chunk_body(i, carry):
            off = base + i * chunk
            pltpu.sync_copy(x_hbm.at[pl.ds(off, chunk)], buf)
            lax.fori_loop(0, chunk // 128, fix_row, 0)
            pltpu.sync_copy(buf, o_hbm.at[pl.ds(off, chunk)])
            return carry

        lax.fori_loop(0, n_chunks, chunk_body, 0)

    return k(x1d)


def kernel(x):
    shape = x.shape
    return _sc_bound(x.reshape(-1)).reshape(shape)


# speedup vs baseline: 1.4383x; 1.4383x over previous
import functools

import jax
import jax.numpy as jnp
import numpy as np
from jax import lax
from jax.experimental import pallas as pl
from jax.experimental.pallas import tpu as pltpu
from jax.experimental.pallas import tpu_sc as plsc

_VARIABLES = ["tp", "cp", "swl"]
_DATA_INDEX = np.array([3, 17, 42], dtype=np.int64)
_MIN_VAL = [0.0, 0.0, 0.0]
_NORMALIZER = ["mean-std", "min-max", "std"]
_STATS = {
    "mean": np.array([0.5, 0.2, 0.1], dtype=np.float32),
    "stdev": np.array([1.2, 0.8, 0.3], dtype=np.float32),
    "min": np.array([0.0, 0.0, 0.0], dtype=np.float32),
    "max": np.array([10.0, 5.0, 1.0], dtype=np.float32),
}
_NAME_TO_INDEX_STATS = {"tp": 0, "cp": 1, "swl": 2}


def _compute_norm_min_val() -> np.ndarray:
    nmv = np.zeros(len(_VARIABLES), dtype=np.float32)
    for ii, var in enumerate(_VARIABLES):
        si = _NAME_TO_INDEX_STATS[var]
        if _NORMALIZER[ii] == "mean-std":
            nmv[ii] = (_MIN_VAL[ii] - _STATS["mean"][si]) / _STATS["stdev"][si]
        elif _NORMALIZER[ii] == "min-max":
            nmv[ii] = (_MIN_VAL[ii] - _STATS["min"][si]) / (_STATS["max"][si] - _STATS["min"][si])
        elif _NORMALIZER[ii] == "max":
            nmv[ii] = _MIN_VAL[ii] / _STATS["max"][si]
        elif _NORMALIZER[ii] == "std":
            nmv[ii] = _MIN_VAL[ii] / _STATS["stdev"][si]
    return nmv


_NMV = _compute_norm_min_val()


def _bound_kernel(x_ref, o_ref):
    lane = jax.lax.broadcasted_iota(jnp.int32, (1, 128), 1)
    lmin = jnp.full((1, 128), -jnp.inf, jnp.float32)
    for c, v in zip(_DATA_INDEX, _NMV):
        lmin = jnp.where(lane == int(c), jnp.float32(v), lmin)
    o_ref[...] = jnp.maximum(x_ref[...], lmin)


@functools.partial(jax.jit, static_argnames=("block_rows",))
def _bound(x2d, block_rows):
    rows = x2d.shape[0]
    return pl.pallas_call(
        _bound_kernel,
        grid=(rows // block_rows,),
        in_specs=[pl.BlockSpec((block_rows, 128), lambda i: (i, 0))],
        out_specs=pl.BlockSpec((block_rows, 128), lambda i: (i, 0)),
        out_shape=jax.ShapeDtypeStruct(x2d.shape, x2d.dtype),
    )(x2d)


@jax.jit
def _sc_bound(x1d):
    info = plsc.get_sparse_core_info()
    nc, ns = info.num_cores, info.num_subcores
    nw = nc * ns
    total = x1d.shape[0]
    per_w = total // nw
    chunk = 32768
    n_chunks = per_w // chunk
    mesh = plsc.VectorSubcoreMesh(core_axis_name="c", subcore_axis_name="s")

    @functools.partial(
        pl.kernel,
        mesh=mesh,
        out_type=jax.ShapeDtypeStruct((total,), jnp.float32),
        scratch_types=[
            pltpu.VMEM((chunk,), jnp.float32),
            pltpu.VMEM((chunk,), jnp.float32),
            pltpu.SemaphoreType.DMA,
            pltpu.SemaphoreType.DMA,
            pltpu.SemaphoreType.DMA,
            pltpu.SemaphoreType.DMA,
        ],
    )
    def k(x_hbm, o_hbm, buf0, buf1, si0, si1, so0, so1):
        wid = lax.axis_index("s") * nc + lax.axis_index("c")
        base = wid * per_w
        bufs, sin, sout = (buf0, buf1), (si0, si1), (so0, so1)
        lane = lax.iota(jnp.int32, 16)
        ninf = jnp.float32(-jnp.inf)
        floors = [
            (0, jnp.where(lane == 3, jnp.float32(_NMV[0]), ninf)),
            (16, jnp.where(lane == 1, jnp.float32(_NMV[1]), ninf)),
            (32, jnp.where(lane == 10, jnp.float32(_NMV[2]), ninf)),
        ]

        def fix_rows(buf):
            def fix_row(r, carry):
                row = r * 128
                for off, fl in floors:
                    sl = pl.ds(row + off, 16)
                    buf[sl] = jnp.maximum(buf[sl], fl)
                return carry

            lax.fori_loop(0, chunk // 128, fix_row, 0)

        def copy_in(g):
            b = g % 2
            return pltpu.make_async_copy(
                x_hbm.at[pl.ds(base + g * chunk, chunk)], bufs[b], sin[b])

        def copy_out(g):
            b = g % 2
            return pltpu.make_async_copy(
                bufs[b], o_hbm.at[pl.ds(base + g * chunk, chunk)], sout[b])

        copy_in(0).start()
        for g in range(n_chunks):
            if g + 1 < n_chunks:
                if g >= 1:
                    copy_out(g - 1).wait()
                copy_in(g + 1).start()
            copy_in(g).wait()
            fix_rows(bufs[g % 2])
            copy_out(g).start()
        copy_out(n_chunks - 2).wait()
        copy_out(n_chunks - 1).wait()

    return k(x1d)


def kernel(x):
    shape = x.shape
    return _sc_bound(x.reshape(-1)).reshape(shape)
